# Initial kernel scaffold; baseline (speedup 1.0000x reference)
import jax
import jax.numpy as jnp
from jax.experimental import pallas as pl


def kernel(x, W_in, b_in, W_out, b_out):
    raise NotImplementedError("write your pallas kernel here")


# speedup vs baseline: 1.5531x; 1.5531x over previous
import functools

import jax
import jax.numpy as jnp
import numpy as np
from jax import lax
from jax.experimental import pallas as pl
from jax.experimental.pallas import tpu as pltpu
from jax.experimental.pallas import tpu_sc as plsc

_G = 2
_R = 2
_DIM = 1024
_DPG = _DIM // _G
_CD = 4
_NIND = 625
_NBINS = 640
_EPS = 1e-5
_HALF_L = 4.0 * (1.0 + 1e-3) / 2.0
_TB = 512

_NW = 32
_CHUNK = 1024


def _fsq_tc_body(x_ref, wi_ref, bi_ref, woa_ref, feat_ref, i0_ref, i1_ref):
    xb = x_ref[0]
    wi = wi_ref[0]
    bi = bi_ref[0]
    h = jnp.dot(wi, xb, preferred_element_type=jnp.float32) + bi

    def stage(res, scale_inv, scale):
        q = jnp.round(jnp.tanh(res * scale_inv) * _HALF_L)
        idx = ((q[3] * 5.0 + q[2]) * 5.0 + q[1]) * 5.0 + q[0] + 312.0
        return q * (0.5 * scale), idx

    quant0, idx0 = stage(h, 1.0, 1.0)
    quant1, idx1 = stage(h - quant0, 4.0, 0.25)
    qout = quant0 + quant1

    woa = woa_ref[0]
    ones = jnp.ones((1, qout.shape[1]), dtype=jnp.float32)
    qaug = jnp.concatenate([qout, ones], axis=0)
    feat_ref[0] = jnp.dot(woa, qaug, preferred_element_type=jnp.float32)
    i0_ref[0, 0] = idx0.astype(jnp.int32)
    i1_ref[0, 0] = idx1.astype(jnp.int32)


def _fsq_tc(x, w_in, b_in, w_out_aug, t_size):
    b, _, t = x.shape
    nt = t // _TB
    grid = (_G, b, nt)
    out_shapes = (
        jax.ShapeDtypeStruct((b, _DIM, t), jnp.float32),
        jax.ShapeDtypeStruct((_G * b, 1, t), jnp.int32),
        jax.ShapeDtypeStruct((_G * b, 1, t), jnp.int32),
    )
    return pl.pallas_call(
        _fsq_tc_body,
        grid=grid,
        in_specs=[
            pl.BlockSpec((1, _DPG, _TB), lambda g, bb, tt: (bb, g, tt)),
            pl.BlockSpec((1, _CD, _DPG), lambda g, bb, tt: (g, 0, 0)),
            pl.BlockSpec((1, _CD, 1), lambda g, bb, tt: (g, 0, 0)),
            pl.BlockSpec((1, _DPG, _CD + 1), lambda g, bb, tt: (g, 0, 0)),
        ],
        out_specs=(
            pl.BlockSpec((1, _DPG, _TB), lambda g, bb, tt: (bb, g, tt)),
            pl.BlockSpec((1, 1, _TB), lambda g, bb, tt: (g * 4 + bb, 0, tt)),
            pl.BlockSpec((1, 1, _TB), lambda g, bb, tt: (g * 4 + bb, 0, tt)),
        ),
        out_shape=out_shapes,
        compiler_params=pltpu.CompilerParams(
            dimension_semantics=("arbitrary", "arbitrary", "arbitrary"),
        ),
    )(x, w_in, b_in, w_out_aug)


def _hist_sc_body(ind_hbm, out_hbm, idx_v, hist_v, acc_v):
    c = lax.axis_index("c")
    s = lax.axis_index("s")
    w = c * 16 + s

    pltpu.sync_copy(ind_hbm.at[pl.ds(w * _CHUNK, _CHUNK)], idx_v)

    zeros16 = jnp.zeros((16,), dtype=jnp.float32)
    lanes = lax.iota(jnp.int32, 16)

    def zero_body(i, carry):
        hist_v[pl.ds(i * 16, 16)] = zeros16
        return carry

    lax.fori_loop(0, 16 * _NBINS // 16, zero_body, 0)

    ones16 = jnp.ones((16,), dtype=jnp.float32)
    lane_off = lanes * _NBINS

    def scat_body(i, carry):
        v = idx_v[pl.ds(i * 16, 16)]
        plsc.addupdate_scatter(hist_v, [lane_off + v], ones16)
        return carry

    lax.fori_loop(0, _CHUNK // 16, scat_body, 0)

    def red_body(j, carry):
        tot = hist_v[pl.ds(j * 16, 16)]
        for l in range(1, 16):
            tot = tot + hist_v[pl.ds(l * _NBINS + j * 16, 16)]
        acc_v[pl.ds(j * 16, 16)] = tot
        return carry

    lax.fori_loop(0, _NBINS // 16, red_body, 0)

    pltpu.sync_copy(acc_v, out_hbm.at[pl.ds(w * _NBINS, _NBINS)])


def _hist_partial(ind_flat):
    mesh = plsc.VectorSubcoreMesh(core_axis_name="c", subcore_axis_name="s")
    k = functools.partial(
        pl.kernel,
        mesh=mesh,
        out_type=jax.ShapeDtypeStruct((_NW * _NBINS,), jnp.float32),
        scratch_types=[
            pltpu.VMEM((_CHUNK,), jnp.int32),
            pltpu.VMEM((16 * _NBINS,), jnp.float32),
            pltpu.VMEM((_NBINS,), jnp.float32),
        ],
        compiler_params=pltpu.CompilerParams(needs_layout_passes=False),
    )(_hist_sc_body)
    return k(ind_flat)


def _stats_tc_body(c_ref, p_ref):
    c32 = c_ref[...]
    denom = jnp.float32(1.0 / 8192.0)
    plx = []
    for gr in range(4):
        g, r = gr // 2, gr % 2
        lo = r * 16 + g * 8
        cnt = jnp.sum(c32[lo:lo + 8], axis=0)
        e = cnt * denom
        ssum = jnp.sum(e)
        p = e / (ssum + _EPS)
        plx.append(jnp.exp(-jnp.sum(p * jnp.log(p + _EPS))))
    p_ref[0, 0:4] = jnp.stack(plx)


def _stats_tc(c32):
    return pl.pallas_call(
        _stats_tc_body,
        out_shape=jax.ShapeDtypeStruct((1, 4), jnp.float32),
    )(c32)


def kernel(x, W_in, b_in, W_out, b_out):
    b, d, t = x.shape
    w_out_aug = jnp.concatenate([W_out, b_out[:, :, None]], axis=2)
    bi = b_in[:, :, None]

    feat, i0, i1 = _fsq_tc(x, W_in, bi, w_out_aug, t)

    ind_flat = jnp.concatenate(
        [i0.reshape(_G * b * t), i1.reshape(_G * b * t)], axis=0)
    c32 = _hist_partial(ind_flat).reshape(_NW, _NBINS)
    perp = _stats_tc(c32)[0]

    ir = jnp.stack([i0.reshape(_G, b, t), i1.reshape(_G, b, t)], axis=1)
    ind_out = jnp.transpose(ir, (2, 0, 1, 3)).reshape(b, _G * _R, t)

    zeros = jnp.zeros_like(perp)
    return zeros, feat, perp, ind_out


# speedup vs baseline: 1.8633x; 1.1997x over previous
import functools

import jax
import jax.numpy as jnp
import numpy as np
from jax import lax
from jax.experimental import pallas as pl
from jax.experimental.pallas import tpu as pltpu
from jax.experimental.pallas import tpu_sc as plsc

_G = 2
_R = 2
_DIM = 1024
_DPG = _DIM // _G
_CD = 4
_NIND = 625
_NBINS = 640
_EPS = 1e-5
_HALF_L = 4.0 * (1.0 + 1e-3) / 2.0
_TB = 1024

_NW = 32
_CHUNK = 1024


def _fsq_tc_body(x_ref, wi_ref, bi_ref, woa_ref, feat_ref, i0_ref, i1_ref):
    xb = x_ref[0]
    wi = wi_ref[0]
    bi = bi_ref[0]
    h = jnp.dot(wi, xb, preferred_element_type=jnp.float32) + bi

    def stage(res, scale_inv, scale):
        q = jnp.round(jnp.tanh(res * scale_inv) * _HALF_L)
        idx = ((q[3] * 5.0 + q[2]) * 5.0 + q[1]) * 5.0 + q[0] + 312.0
        return q * (0.5 * scale), idx

    quant0, idx0 = stage(h, 1.0, 1.0)
    quant1, idx1 = stage(h - quant0, 4.0, 0.25)
    qout = quant0 + quant1

    woa = woa_ref[0]
    ones = jnp.ones((1, qout.shape[1]), dtype=jnp.float32)
    qaug = jnp.concatenate([qout, ones], axis=0)
    feat_ref[0] = jnp.dot(woa, qaug, preferred_element_type=jnp.float32)
    i0_ref[0, 0] = idx0.astype(jnp.int32)
    i1_ref[0, 0] = idx1.astype(jnp.int32)


def _fsq_tc(x, w_in, b_in, w_out_aug, t_size):
    b, _, t = x.shape
    nt = t // _TB
    grid = (_G, b, nt)
    out_shapes = (
        jax.ShapeDtypeStruct((b, _DIM, t), jnp.float32),
        jax.ShapeDtypeStruct((_G * b, 1, t), jnp.int32),
        jax.ShapeDtypeStruct((_G * b, 1, t), jnp.int32),
    )
    return pl.pallas_call(
        _fsq_tc_body,
        grid=grid,
        in_specs=[
            pl.BlockSpec((1, _DPG, _TB), lambda g, bb, tt: (bb, g, tt)),
            pl.BlockSpec((1, _CD, _DPG), lambda g, bb, tt: (g, 0, 0)),
            pl.BlockSpec((1, _CD, 1), lambda g, bb, tt: (g, 0, 0)),
            pl.BlockSpec((1, _DPG, _CD + 1), lambda g, bb, tt: (g, 0, 0)),
        ],
        out_specs=(
            pl.BlockSpec((1, _DPG, _TB), lambda g, bb, tt: (bb, g, tt)),
            pl.BlockSpec((1, 1, _TB), lambda g, bb, tt: (g * 4 + bb, 0, tt)),
            pl.BlockSpec((1, 1, _TB), lambda g, bb, tt: (g * 4 + bb, 0, tt)),
        ),
        out_shape=out_shapes,
        compiler_params=pltpu.CompilerParams(
            dimension_semantics=("arbitrary", "arbitrary", "arbitrary"),
        ),
    )(x, w_in, b_in, w_out_aug)


def _hist_sc_body(ind_hbm, out_hbm, idx_v, hist_v, acc_v):
    c = lax.axis_index("c")
    s = lax.axis_index("s")
    w = c * 16 + s

    pltpu.sync_copy(ind_hbm.at[pl.ds(w * _CHUNK, _CHUNK)], idx_v)

    zeros16 = jnp.zeros((16,), dtype=jnp.float32)
    lanes = lax.iota(jnp.int32, 16)

    def zero_body(i, carry):
        hist_v[pl.ds(i * 16, 16)] = zeros16
        return carry

    lax.fori_loop(0, 16 * _NBINS // 16, zero_body, 0)

    ones16 = jnp.ones((16,), dtype=jnp.float32)
    lane_off = lanes * _NBINS

    def scat_body(i, carry):
        v = idx_v[pl.ds(i * 16, 16)]
        plsc.addupdate_scatter(hist_v, [lane_off + v], ones16)
        return carry

    lax.fori_loop(0, _CHUNK // 16, scat_body, 0)

    def red_body(j, carry):
        tot = hist_v[pl.ds(j * 16, 16)]
        for l in range(1, 16):
            tot = tot + hist_v[pl.ds(l * _NBINS + j * 16, 16)]
        acc_v[pl.ds(j * 16, 16)] = tot
        return carry

    lax.fori_loop(0, _NBINS // 16, red_body, 0)

    pltpu.sync_copy(acc_v, out_hbm.at[pl.ds(w * _NBINS, _NBINS)])


def _hist_partial(ind_flat):
    mesh = plsc.VectorSubcoreMesh(core_axis_name="c", subcore_axis_name="s")
    k = functools.partial(
        pl.kernel,
        mesh=mesh,
        out_type=jax.ShapeDtypeStruct((_NW * _NBINS,), jnp.float32),
        scratch_types=[
            pltpu.VMEM((_CHUNK,), jnp.int32),
            pltpu.VMEM((16 * _NBINS,), jnp.float32),
            pltpu.VMEM((_NBINS,), jnp.float32),
        ],
        compiler_params=pltpu.CompilerParams(needs_layout_passes=False),
    )(_hist_sc_body)
    return k(ind_flat)


def _stats_tc_body(c_ref, p_ref):
    c32 = c_ref[...]
    denom = jnp.float32(1.0 / 8192.0)
    plx = []
    for gr in range(4):
        g, r = gr // 2, gr % 2
        lo = r * 16 + g * 8
        cnt = jnp.sum(c32[lo:lo + 8], axis=0)
        e = cnt * denom
        ssum = jnp.sum(e)
        p = e / (ssum + _EPS)
        plx.append(jnp.exp(-jnp.sum(p * jnp.log(p + _EPS))))
    p_ref[0, 0:4] = jnp.stack(plx)


def _stats_tc(c32):
    return pl.pallas_call(
        _stats_tc_body,
        out_shape=jax.ShapeDtypeStruct((1, 4), jnp.float32),
    )(c32)


def kernel(x, W_in, b_in, W_out, b_out):
    b, d, t = x.shape
    w_out_aug = jnp.concatenate([W_out, b_out[:, :, None]], axis=2)
    bi = b_in[:, :, None]

    feat, i0, i1 = _fsq_tc(x, W_in, bi, w_out_aug, t)

    ind_flat = jnp.concatenate(
        [i0.reshape(_G * b * t), i1.reshape(_G * b * t)], axis=0)
    c32 = _hist_partial(ind_flat).reshape(_NW, _NBINS)
    perp = _stats_tc(c32)[0]

    ir = jnp.stack([i0.reshape(_G, b, t), i1.reshape(_G, b, t)], axis=1)
    ind_out = jnp.transpose(ir, (2, 0, 1, 3)).reshape(b, _G * _R, t)

    zeros = jnp.zeros_like(perp)
    return zeros, feat, perp, ind_out


# speedup vs baseline: 1.9928x; 1.0695x over previous
import functools

import jax
import jax.numpy as jnp
import numpy as np
from jax import lax
from jax.experimental import pallas as pl
from jax.experimental.pallas import tpu as pltpu
from jax.experimental.pallas import tpu_sc as plsc

_G = 2
_R = 2
_DIM = 1024
_DPG = _DIM // _G
_CD = 4
_NIND = 625
_NBINS = 640
_EPS = 1e-5
_HALF_L = 4.0 * (1.0 + 1e-3) / 2.0
_TB = 2048

_NW = 32
_CHUNK = 1024


def _fsq_tc_body(x_ref, wi_ref, bi_ref, woa_ref, feat_ref, i0_ref, i1_ref):
    xb = x_ref[0]
    wi = wi_ref[0]
    bi = bi_ref[0]
    h = jnp.dot(wi, xb, preferred_element_type=jnp.float32) + bi

    def stage(res, scale_inv, scale):
        q = jnp.round(jnp.tanh(res * scale_inv) * _HALF_L)
        idx = ((q[3] * 5.0 + q[2]) * 5.0 + q[1]) * 5.0 + q[0] + 312.0
        return q * (0.5 * scale), idx

    quant0, idx0 = stage(h, 1.0, 1.0)
    quant1, idx1 = stage(h - quant0, 4.0, 0.25)
    qout = quant0 + quant1

    woa = woa_ref[0]
    ones = jnp.ones((1, qout.shape[1]), dtype=jnp.float32)
    qaug = jnp.concatenate([qout, ones], axis=0)
    feat_ref[0] = jnp.dot(woa, qaug, preferred_element_type=jnp.float32)
    i0_ref[0, 0] = idx0.astype(jnp.int32)
    i1_ref[0, 0] = idx1.astype(jnp.int32)


def _fsq_tc(x, w_in, b_in, w_out_aug, t_size):
    b, _, t = x.shape
    nt = t // _TB
    grid = (_G, b, nt)
    out_shapes = (
        jax.ShapeDtypeStruct((b, _DIM, t), jnp.float32),
        jax.ShapeDtypeStruct((_G * b, 1, t), jnp.int32),
        jax.ShapeDtypeStruct((_G * b, 1, t), jnp.int32),
    )
    return pl.pallas_call(
        _fsq_tc_body,
        grid=grid,
        in_specs=[
            pl.BlockSpec((1, _DPG, _TB), lambda g, bb, tt: (bb, g, tt)),
            pl.BlockSpec((1, _CD, _DPG), lambda g, bb, tt: (g, 0, 0)),
            pl.BlockSpec((1, _CD, 1), lambda g, bb, tt: (g, 0, 0)),
            pl.BlockSpec((1, _DPG, _CD + 1), lambda g, bb, tt: (g, 0, 0)),
        ],
        out_specs=(
            pl.BlockSpec((1, _DPG, _TB), lambda g, bb, tt: (bb, g, tt)),
            pl.BlockSpec((1, 1, _TB), lambda g, bb, tt: (g * 4 + bb, 0, tt)),
            pl.BlockSpec((1, 1, _TB), lambda g, bb, tt: (g * 4 + bb, 0, tt)),
        ),
        out_shape=out_shapes,
        compiler_params=pltpu.CompilerParams(
            dimension_semantics=("arbitrary", "arbitrary", "arbitrary"),
        ),
    )(x, w_in, b_in, w_out_aug)


def _hist_sc_body(ind_hbm, out_hbm, idx_v, hist_v, acc_v):
    c = lax.axis_index("c")
    s = lax.axis_index("s")
    w = c * 16 + s

    pltpu.sync_copy(ind_hbm.at[pl.ds(w * _CHUNK, _CHUNK)], idx_v)

    zeros16 = jnp.zeros((16,), dtype=jnp.float32)
    lanes = lax.iota(jnp.int32, 16)

    def zero_body(i, carry):
        hist_v[pl.ds(i * 16, 16)] = zeros16
        return carry

    lax.fori_loop(0, 16 * _NBINS // 16, zero_body, 0)

    ones16 = jnp.ones((16,), dtype=jnp.float32)
    lane_off = lanes * _NBINS

    def scat_body(i, carry):
        v = idx_v[pl.ds(i * 16, 16)]
        plsc.addupdate_scatter(hist_v, [lane_off + v], ones16)
        return carry

    lax.fori_loop(0, _CHUNK // 16, scat_body, 0)

    def red_body(j, carry):
        tot = hist_v[pl.ds(j * 16, 16)]
        for l in range(1, 16):
            tot = tot + hist_v[pl.ds(l * _NBINS + j * 16, 16)]
        acc_v[pl.ds(j * 16, 16)] = tot
        return carry

    lax.fori_loop(0, _NBINS // 16, red_body, 0)

    pltpu.sync_copy(acc_v, out_hbm.at[pl.ds(w * _NBINS, _NBINS)])


def _hist_partial(ind_flat):
    mesh = plsc.VectorSubcoreMesh(core_axis_name="c", subcore_axis_name="s")
    k = functools.partial(
        pl.kernel,
        mesh=mesh,
        out_type=jax.ShapeDtypeStruct((_NW * _NBINS,), jnp.float32),
        scratch_types=[
            pltpu.VMEM((_CHUNK,), jnp.int32),
            pltpu.VMEM((16 * _NBINS,), jnp.float32),
            pltpu.VMEM((_NBINS,), jnp.float32),
        ],
        compiler_params=pltpu.CompilerParams(needs_layout_passes=False),
    )(_hist_sc_body)
    return k(ind_flat)


def _stats_tc_body(c_ref, p_ref):
    c32 = c_ref[...]
    denom = jnp.float32(1.0 / 8192.0)
    plx = []
    for gr in range(4):
        g, r = gr // 2, gr % 2
        lo = r * 16 + g * 8
        cnt = jnp.sum(c32[lo:lo + 8], axis=0)
        e = cnt * denom
        ssum = jnp.sum(e)
        p = e / (ssum + _EPS)
        plx.append(jnp.exp(-jnp.sum(p * jnp.log(p + _EPS))))
    p_ref[0, 0:4] = jnp.stack(plx)


def _stats_tc(c32):
    return pl.pallas_call(
        _stats_tc_body,
        out_shape=jax.ShapeDtypeStruct((1, 4), jnp.float32),
    )(c32)


def kernel(x, W_in, b_in, W_out, b_out):
    b, d, t = x.shape
    w_out_aug = jnp.concatenate([W_out, b_out[:, :, None]], axis=2)
    bi = b_in[:, :, None]

    feat, i0, i1 = _fsq_tc(x, W_in, bi, w_out_aug, t)

    ind_flat = jnp.concatenate(
        [i0.reshape(_G * b * t), i1.reshape(_G * b * t)], axis=0)
    c32 = _hist_partial(ind_flat).reshape(_NW, _NBINS)
    perp = _stats_tc(c32)[0]

    ir = jnp.stack([i0.reshape(_G, b, t), i1.reshape(_G, b, t)], axis=1)
    ind_out = jnp.transpose(ir, (2, 0, 1, 3)).reshape(b, _G * _R, t)

    zeros = jnp.zeros_like(perp)
    return zeros, feat, perp, ind_out
